# Initial kernel scaffold; baseline (speedup 1.0000x reference)
import jax
import jax.numpy as jnp
from jax.experimental import pallas as pl


def kernel(x, edge_index, node_type, edge_type, edge_attr, batch, params):
    raise NotImplementedError("write your pallas kernel here")


# speedup vs baseline: 2.8049x; 2.8049x over previous
import functools

import jax
import jax.numpy as jnp
from jax import lax
from jax.experimental import pallas as pl
from jax.experimental.pallas import tpu as pltpu
from jax.experimental.pallas import tpu_sc as plsc

N = 10000
E = 160000
IN = 128
HID = 128
OUT = 64
ED = 16
T = 8
ET = 16
ETD = 32
EAD = 32
G = 64
NEG = 0.2

NC = 2
NS = 16
EW = 10240
E_PAD = NS * EW
C1 = 128
N1 = EW // C1
C2 = 64
N2 = EW // C2
NP = 10240
NPT = NP // NS
NROW = NP // NS
S2W = 48


def _lrelu(x):
    return jnp.where(x >= 0, x, NEG * x)


def _node_body(x_ref, nt_ref, wh_ref, bh_ref, wij_ref, h_ref, sij_ref):
    x = x_ref[...]
    nt = nt_ref[...]
    acc = jnp.zeros((x.shape[0], HID), jnp.float32)
    for t in range(T):
        m = (nt == t).astype(jnp.float32)
        acc = acc + m * (
            jnp.dot(x, wh_ref[t], preferred_element_type=jnp.float32)
            + bh_ref[t][None, :])
    h_ref[...] = acc
    sij_ref[...] = jnp.dot(acc, wij_ref[...],
                           preferred_element_type=jnp.float32)


def _node_tc(x, nt2, wh, bh, wij):
    return pl.pallas_call(
        _node_body,
        out_shape=[jax.ShapeDtypeStruct((N, HID), jnp.float32),
                   jax.ShapeDtypeStruct((N, 2), jnp.float32)],
    )(x, nt2, wh, bh, wij)


def _edge_body(ea_ref, et_ref, wea_ref, eemb_ref, wa_ea_ref, wa_et_ref,
               ea_out_ref, t_out_ref):
    a = ea_ref[...]
    ea = _lrelu(jnp.dot(a, wea_ref[...],
                        preferred_element_type=jnp.float32))
    ea_out_ref[...] = ea
    c = jnp.dot(_lrelu(eemb_ref[...]), wa_et_ref[...],
                preferred_element_type=jnp.float32)
    et = et_ref[...]
    onehot = (et == lax.broadcasted_iota(jnp.int32, (et.shape[0], ET), 1)
              ).astype(jnp.float32)
    cterm = jnp.dot(onehot, c, preferred_element_type=jnp.float32)
    t_out_ref[...] = cterm + jnp.dot(ea, wa_ea_ref[...],
                                     preferred_element_type=jnp.float32)


def _edge_tc(ea_in, et2, wea, eemb, wa_ea, wa_et):
    nblk = 20
    be = E // nblk
    return pl.pallas_call(
        _edge_body,
        grid=(nblk,),
        in_specs=[
            pl.BlockSpec((be, ED), lambda i: (i, 0)),
            pl.BlockSpec((be, 1), lambda i: (i, 0)),
            pl.BlockSpec((ED, EAD), lambda i: (0, 0)),
            pl.BlockSpec((ET, ETD), lambda i: (0, 0)),
            pl.BlockSpec((EAD, 1), lambda i: (0, 0)),
            pl.BlockSpec((ETD, 1), lambda i: (0, 0)),
        ],
        out_specs=[
            pl.BlockSpec((be, EAD), lambda i: (i, 0)),
            pl.BlockSpec((be, 1), lambda i: (i, 0)),
        ],
        out_shape=[jax.ShapeDtypeStruct((E, EAD), jnp.float32),
                   jax.ShapeDtypeStruct((E, 1), jnp.float32)],
    )(ea_in, et2, wea, eemb, wa_ea, wa_et)


def _finish_body(h_ref, aggp_ref, s2p_ref, wl1_ref, wl2_ref, bl_ref,
                 o_ref, *, relu):
    agg = aggp_ref[:N, :]
    s2 = s2p_ref[:N, :]
    seg = s2[:, 32:33]
    num = (jnp.dot(agg, wl1_ref[...], preferred_element_type=jnp.float32)
           + jnp.dot(s2[:, :EAD], wl2_ref[...],
                     preferred_element_type=jnp.float32)
           + seg * bl_ref[...])
    o = h_ref[...] + num / (seg + 1e-16)
    if relu:
        o = jnp.maximum(o, 0.0)
    o_ref[...] = o


def _finish_tc(h, aggp, s2p, wl1, wl2, bl2, relu):
    return pl.pallas_call(
        functools.partial(_finish_body, relu=relu),
        out_shape=jax.ShapeDtypeStruct((N, HID), jnp.float32),
    )(h, aggp, s2p, wl1, wl2, bl2)


def _pool_body(emb_ref, b_ref, w1_ref, b1_ref, w2_ref, b2_ref, out_ref):
    hr = jnp.maximum(emb_ref[...], 0.0)
    b = b_ref[...]
    onehot = (b == lax.broadcasted_iota(jnp.int32, (N, G), 1)
              ).astype(jnp.float32)
    pooled = lax.dot_general(onehot, hr, (((0,), (0,)), ((), ())),
                             preferred_element_type=jnp.float32)
    cnt = jnp.sum(onehot, axis=0)[:, None]
    pooled = pooled / jnp.maximum(cnt, 1.0)
    z = jnp.dot(pooled, w1_ref[...],
                preferred_element_type=jnp.float32) + b1_ref[...]
    out_ref[...] = jnp.dot(z, w2_ref[...],
                           preferred_element_type=jnp.float32) + b2_ref[...]


def _pool_tc(emb, b2, w1, b1, w2, b2b):
    return pl.pallas_call(
        _pool_body,
        out_shape=jax.ShapeDtypeStruct((G, OUT), jnp.float32),
    )(emb, b2, w1, b1, w2, b2b)


def _k1_body(src_ref, dst_ref, t_ref, si_ref, sj_ref,
             aexp_ref,
             si_loc, sj_loc, src2d, dst2d, t2d, a2d, mx, buf, res,
             shared, sem):
    s = lax.axis_index("s")

    pltpu.sync_copy(si_ref, si_loc.at[pl.ds(0, N)])
    pltpu.sync_copy(sj_ref, sj_loc.at[pl.ds(0, N)])
    pltpu.sync_copy(src_ref.at[s], src2d)
    pltpu.sync_copy(dst_ref.at[s], dst2d)
    pltpu.sync_copy(t_ref.at[s], t2d)

    neg = jnp.full((16,), -3.0e38, jnp.float32)

    def init_body(i, _):
        mx[pl.ds(pl.multiple_of(i * 16, 16), 16)] = neg
        return 0
    lax.fori_loop(0, NP // 16, init_body, 0)

    def edge_body(k, _):
        r = k // 8
        c16 = pl.multiple_of((k % 8) * 16, 16)
        srcv = src2d[r, pl.ds(c16, 16)]
        dstv = dst2d[r, pl.ds(c16, 16)]
        sj = plsc.load_gather(sj_loc, [srcv])
        si = plsc.load_gather(si_loc, [dstv])
        a = si + sj + t2d[r, pl.ds(c16, 16)]
        a = jnp.where(a >= 0, a, NEG * a)
        a2d[r, pl.ds(c16, 16)] = a

        def mx_body(carry):
            g = plsc.load_gather(mx, [dstv])
            m = a > g
            plsc.store_scatter(mx, [dstv], a, mask=m)
            g2 = plsc.load_gather(mx, [dstv])
            return jnp.any(a > g2)
        lax.while_loop(lambda cc: cc, mx_body, True)
        return 0
    lax.fori_loop(0, N1 * 8, edge_body, 0)

    pltpu.sync_copy(mx, shared.at[s])
    plsc.subcore_barrier()
    pltpu.sync_copy(shared.at[pl.ds(0, NS), pl.ds(s * NPT, NPT)], buf)

    def red_body(k, _):
        c16 = pl.multiple_of(k * 16, 16)
        m = buf[0, pl.ds(c16, 16)]
        for rr in range(1, NS):
            m = jnp.maximum(m, buf[rr, pl.ds(c16, 16)])
        res[pl.ds(c16, 16)] = m
        return 0
    lax.fori_loop(0, NPT // 16, red_body, 0)
    pltpu.sync_copy(res, shared.at[NS, pl.ds(s * NPT, NPT)])
    plsc.subcore_barrier()
    pltpu.sync_copy(shared.at[NS], mx)

    def aexp_body(k, _):
        r = k // 8
        c16 = pl.multiple_of((k % 8) * 16, 16)
        dstv = dst2d[r, pl.ds(c16, 16)]
        av = a2d[r, pl.ds(c16, 16)]
        am = plsc.load_gather(mx, [dstv])
        a2d[r, pl.ds(c16, 16)] = jnp.exp(av - am)
        return 0
    lax.fori_loop(0, N1 * 8, aexp_body, 0)
    pltpu.sync_copy(a2d, aexp_ref.at[s])


def _k1_sc(src3, dst3, t3, s_i, s_j):
    mesh = plsc.VectorSubcoreMesh(core_axis_name="c", subcore_axis_name="s",
                                  num_cores=1)
    return pl.kernel(
        _k1_body,
        out_type=jax.ShapeDtypeStruct((NS, N1, C1), jnp.float32),
        mesh=mesh,
        scratch_types=[
            pltpu.VMEM((NP,), jnp.float32),
            pltpu.VMEM((NP,), jnp.float32),
            pltpu.VMEM((N1, C1), jnp.int32),
            pltpu.VMEM((N1, C1), jnp.int32),
            pltpu.VMEM((N1, C1), jnp.float32),
            pltpu.VMEM((N1, C1), jnp.float32),
            pltpu.VMEM((NP,), jnp.float32),
            pltpu.VMEM((NS, NPT), jnp.float32),
            pltpu.VMEM((NPT,), jnp.float32),
            pltpu.VMEM_SHARED((NS + 1, NP), jnp.float32),
            pltpu.SemaphoreType.DMA,
        ],
        compiler_params=pltpu.CompilerParams(needs_layout_passes=False,
                                             use_tc_tiling_on_sc=False),
    )(src3, dst3, t3, s_i, s_j)


def _k2_body(src_ref, dst_ref, ax_ref, h_ref, ea_ref,
             aggp_ref, s2p_ref,
             src1, dst_c, ax1, hb, ea_buf, s2_buf, agg_sh, s2_sh, sem):
    wid = lax.axis_index("s")

    zv = jnp.zeros((16,), jnp.float32)

    def z_body(i, _):
        r = i // 8
        o = pl.multiple_of((i % 8) * 16, 16)
        hb[r, pl.ds(o, 16)] = zv
        return 0
    lax.fori_loop(0, C2 * 8, z_body, 0)

    def z2_body(i, _):
        r = i // 3
        o = pl.multiple_of((i % 3) * 16, 16)
        s2_buf[r, pl.ds(o, 16)] = zv
        return 0
    lax.fori_loop(0, C2 * 3, z2_body, 0)

    for kk in range(NROW // C2):
        row = wid * NROW + kk * C2
        pltpu.sync_copy(hb, agg_sh.at[pl.ds(row, C2)])
        pltpu.sync_copy(s2_buf, s2_sh.at[pl.ds(row, C2)])
    plsc.subcore_barrier()

    lane0 = (lax.iota(jnp.int32, 16) == 0).astype(jnp.float32)

    def chunk_body(j, _):
        pltpu.sync_copy(src_ref.at[wid, j], src1)
        pltpu.sync_copy(dst_ref.at[wid, j], dst_c.at[0])
        pltpu.sync_copy(ax_ref.at[wid, j], ax1.at[pl.ds(0, C2)])
        pltpu.async_copy(h_ref.at[src1], hb, sem).wait()
        pltpu.sync_copy(ea_ref.at[wid, j], ea_buf)

        def scale_body(i, _):
            sc = ax1[pl.ds(i, 16)][0]
            for q in range(8):
                o = pl.multiple_of(q * 16, 16)
                hb[i, pl.ds(o, 16)] = hb[i, pl.ds(o, 16)] * sc
            s2_buf[i, pl.ds(0, 16)] = ea_buf[i, pl.ds(0, 16)] * sc
            s2_buf[i, pl.ds(16, 16)] = ea_buf[i, pl.ds(16, 16)] * sc
            s2_buf[i, pl.ds(32, 16)] = lane0 * sc
            return 0
        lax.fori_loop(0, C2, scale_body, 0)

        pltpu.sync_copy(hb, agg_sh.at[dst_c.at[0]], add=True)
        pltpu.sync_copy(s2_buf, s2_sh.at[dst_c.at[0]], add=True)
        return 0
    lax.fori_loop(0, N2, chunk_body, 0)

    plsc.subcore_barrier()
    row = wid * NROW
    pltpu.sync_copy(agg_sh.at[pl.ds(row, NROW)],
                    aggp_ref.at[pl.ds(row, NROW)])
    pltpu.sync_copy(s2_sh.at[pl.ds(row, NROW)],
                    s2p_ref.at[pl.ds(row, NROW)])


def _k2_sc(src3, dst3, aexp3, h, ea4):
    mesh = plsc.VectorSubcoreMesh(core_axis_name="c", subcore_axis_name="s",
                                  num_cores=1)
    return pl.kernel(
        _k2_body,
        out_type=[
            jax.ShapeDtypeStruct((NP, HID), jnp.float32),
            jax.ShapeDtypeStruct((NP, S2W), jnp.float32),
        ],
        mesh=mesh,
        scratch_types=[
            pltpu.VMEM((C2,), jnp.int32),
            pltpu.VMEM((1, C2), jnp.int32),
            pltpu.VMEM((C2 + 16,), jnp.float32),
            pltpu.VMEM((C2, HID), jnp.float32),
            pltpu.VMEM((C2, EAD), jnp.float32),
            pltpu.VMEM((C2, S2W), jnp.float32),
            pltpu.VMEM_SHARED((NP, HID), jnp.float32),
            pltpu.VMEM_SHARED((NP, S2W), jnp.float32),
            pltpu.SemaphoreType.DMA,
        ],
        compiler_params=pltpu.CompilerParams(needs_layout_passes=False,
                                             use_tc_tiling_on_sc=False),
    )(src3, dst3, aexp3, h, ea4)


def _layer(x, src3, dst3, nt2, et2, edge_attr, p, relu):
    wij = p['Watt'][:2 * HID, :].reshape(2, HID, 1)[..., 0].T
    h, sij = _node_tc(x, nt2, p['Wh'], p['bh'], wij)
    ea, t = _edge_tc(edge_attr, et2, p['Wea'], p['Eemb'],
                     p['Watt'][2 * HID + ETD:, :], p['Watt'][2 * HID:2 * HID + ETD, :])
    t_p = jnp.concatenate(
        [t[:, 0], jnp.full((E_PAD - E,), -1.0e30, jnp.float32)]
    ).reshape(NS, N1, C1)
    ea4 = jnp.concatenate(
        [ea, jnp.zeros((E_PAD - E, EAD), jnp.float32)]
    ).reshape(NS, N2, C2, EAD)
    aexp3 = _k1_sc(src3, dst3, t_p, sij[:, 0], sij[:, 1])
    aggp, s2p = _k2_sc(src3.reshape(NS, N2, C2),
                       dst3.reshape(NS, N2, C2),
                       aexp3.reshape(NS, N2, C2), h, ea4)
    wl1 = p['Wlin'][:HID, :]
    wl2 = p['Wlin'][HID:, :]
    bl2 = p['blin'][None, :]
    return _finish_tc(h, aggp, s2p, wl1, wl2, bl2, relu)


def kernel(x, edge_index, node_type, edge_type, edge_attr, batch, params):
    src = edge_index[0].astype(jnp.int32)
    dst = edge_index[1].astype(jnp.int32)
    zpad = jnp.zeros((E_PAD - E,), jnp.int32)
    src3 = jnp.concatenate([src, zpad]).reshape(NS, N1, C1)
    dst3 = jnp.concatenate([dst, zpad]).reshape(NS, N1, C1)
    nt2 = node_type.astype(jnp.int32)[:, None]
    et2 = edge_type.astype(jnp.int32)[:, None]
    b2 = batch.astype(jnp.int32)[:, None]

    h = _layer(x, src3, dst3, nt2, et2, edge_attr, params['conv1'], True)
    h = _layer(h, src3, dst3, nt2, et2, edge_attr, params['conv2'], True)
    emb = _layer(h, src3, dst3, nt2, et2, edge_attr, params['conv3'], False)
    out = _pool_tc(emb, b2, params['W1'], params['b1'][None, :],
                   params['W2'], params['b2'][None, :])
    return emb, out


# speedup vs baseline: 3.9793x; 1.4187x over previous
import functools

import jax
import jax.numpy as jnp
from jax import lax
from jax.experimental import pallas as pl
from jax.experimental.pallas import tpu as pltpu
from jax.experimental.pallas import tpu_sc as plsc

N = 10000
E = 160000
IN = 128
HID = 128
OUT = 64
ED = 16
T = 8
ET = 16
ETD = 32
EAD = 32
G = 64
NEG = 0.2

NC = 2
NS = 16
EW = 10240
E_PAD = NS * EW
C1 = 128
N1 = EW // C1
C2 = 64
N2 = EW // C2
NP = 10240
NPT = NP // NS
NROW = NP // NS
S2W = 48


def _lrelu(x):
    return jnp.where(x >= 0, x, NEG * x)


def _node_body(x_ref, nt_ref, wh_ref, bh_ref, wij_ref, h_ref, sij_ref):
    x = x_ref[...]
    nt = nt_ref[...]
    acc = jnp.zeros((x.shape[0], HID), jnp.float32)
    for t in range(T):
        m = (nt == t).astype(jnp.float32)
        acc = acc + m * (
            jnp.dot(x, wh_ref[t], preferred_element_type=jnp.float32)
            + bh_ref[t][None, :])
    h_ref[...] = acc
    sij_ref[...] = jnp.dot(acc, wij_ref[...],
                           preferred_element_type=jnp.float32)


def _node_tc(x, nt2, wh, bh, wij):
    return pl.pallas_call(
        _node_body,
        out_shape=[jax.ShapeDtypeStruct((N, HID), jnp.float32),
                   jax.ShapeDtypeStruct((N, 2), jnp.float32)],
    )(x, nt2, wh, bh, wij)


def _edge_body(ea_ref, et_ref, wea_ref, eemb_ref, wa_ea_ref, wa_et_ref,
               ea_out_ref, t_out_ref):
    a = ea_ref[...]
    ea = _lrelu(jnp.dot(a, wea_ref[...],
                        preferred_element_type=jnp.float32))
    ea_out_ref[...] = ea
    c = jnp.dot(_lrelu(eemb_ref[...]), wa_et_ref[...],
                preferred_element_type=jnp.float32)
    et = et_ref[...]
    onehot = (et == lax.broadcasted_iota(jnp.int32, (et.shape[0], ET), 1)
              ).astype(jnp.float32)
    cterm = jnp.dot(onehot, c, preferred_element_type=jnp.float32)
    t_out_ref[...] = cterm + jnp.dot(ea, wa_ea_ref[...],
                                     preferred_element_type=jnp.float32)


def _edge_tc(ea_in, et2, wea, eemb, wa_ea, wa_et):
    nblk = 20
    be = E // nblk
    return pl.pallas_call(
        _edge_body,
        grid=(nblk,),
        in_specs=[
            pl.BlockSpec((be, ED), lambda i: (i, 0)),
            pl.BlockSpec((be, 1), lambda i: (i, 0)),
            pl.BlockSpec((ED, EAD), lambda i: (0, 0)),
            pl.BlockSpec((ET, ETD), lambda i: (0, 0)),
            pl.BlockSpec((EAD, 1), lambda i: (0, 0)),
            pl.BlockSpec((ETD, 1), lambda i: (0, 0)),
        ],
        out_specs=[
            pl.BlockSpec((be, EAD), lambda i: (i, 0)),
            pl.BlockSpec((be, 1), lambda i: (i, 0)),
        ],
        out_shape=[jax.ShapeDtypeStruct((E, EAD), jnp.float32),
                   jax.ShapeDtypeStruct((E, 1), jnp.float32)],
    )(ea_in, et2, wea, eemb, wa_ea, wa_et)


def _finish_body(h_ref, aggp_ref, s2p_ref, wl1_ref, wl2_ref, bl_ref,
                 o_ref, *, relu):
    agg = aggp_ref[...]
    s2 = s2p_ref[...]
    seg = s2[:, 32:33]
    num = (jnp.dot(agg, wl1_ref[...], preferred_element_type=jnp.float32)
           + jnp.dot(s2[:, :EAD], wl2_ref[...],
                     preferred_element_type=jnp.float32)
           + seg * bl_ref[...])
    o = h_ref[...] + num / (seg + 1e-16)
    if relu:
        o = jnp.maximum(o, 0.0)
    o_ref[...] = o


def _finish_tc(h, aggp, s2p, wl1, wl2, bl2, relu):
    return pl.pallas_call(
        functools.partial(_finish_body, relu=relu),
        out_shape=jax.ShapeDtypeStruct((N, HID), jnp.float32),
    )(h, aggp, s2p, wl1, wl2, bl2)


def _pool_body(emb_ref, b_ref, w1_ref, b1_ref, w2_ref, b2_ref, out_ref):
    hr = jnp.maximum(emb_ref[...], 0.0)
    b = b_ref[...]
    onehot = (b == lax.broadcasted_iota(jnp.int32, (N, G), 1)
              ).astype(jnp.float32)
    pooled = lax.dot_general(onehot, hr, (((0,), (0,)), ((), ())),
                             preferred_element_type=jnp.float32)
    cnt = jnp.sum(onehot, axis=0)[:, None]
    pooled = pooled / jnp.maximum(cnt, 1.0)
    z = jnp.dot(pooled, w1_ref[...],
                preferred_element_type=jnp.float32) + b1_ref[...]
    out_ref[...] = jnp.dot(z, w2_ref[...],
                           preferred_element_type=jnp.float32) + b2_ref[...]


def _pool_tc(emb, b2, w1, b1, w2, b2b):
    return pl.pallas_call(
        _pool_body,
        out_shape=jax.ShapeDtypeStruct((G, OUT), jnp.float32),
    )(emb, b2, w1, b1, w2, b2b)


def _k1_body(src_ref, dst_ref, t_ref, si_ref, sj_ref, ea_ref,
             aexp_ref, s2p_ref,
             si_loc, sj_loc, src2d, dst2d, t2d, a2d, mx, buf, res,
             ax_pad, ea_buf, s2_buf, shared, s2_sh, sem):
    s = lax.axis_index("s")

    pltpu.sync_copy(si_ref, si_loc.at[pl.ds(0, N)])
    pltpu.sync_copy(sj_ref, sj_loc.at[pl.ds(0, N)])
    pltpu.sync_copy(src_ref.at[s], src2d)
    pltpu.sync_copy(dst_ref.at[s], dst2d)
    pltpu.sync_copy(t_ref.at[s], t2d)

    zv = jnp.zeros((16,), jnp.float32)

    def z2_body(i, _):
        r = i // 3
        o = pl.multiple_of((i % 3) * 16, 16)
        s2_buf[r, pl.ds(o, 16)] = zv
        return 0
    lax.fori_loop(0, C1 * 3, z2_body, 0)
    for kk in range(5):
        row = s * 625 + kk * 125
        pltpu.sync_copy(s2_buf.at[pl.ds(0, 125)], s2_sh.at[pl.ds(row, 125)])

    neg = jnp.full((16,), -3.0e38, jnp.float32)

    def init_body(i, _):
        mx[pl.ds(pl.multiple_of(i * 16, 16), 16)] = neg
        return 0
    lax.fori_loop(0, NP // 16, init_body, 0)

    def edge_body(k, _):
        r = k // 8
        c16 = pl.multiple_of((k % 8) * 16, 16)
        srcv = src2d[r, pl.ds(c16, 16)]
        dstv = dst2d[r, pl.ds(c16, 16)]
        sj = plsc.load_gather(sj_loc, [srcv])
        si = plsc.load_gather(si_loc, [dstv])
        a = si + sj + t2d[r, pl.ds(c16, 16)]
        a = jnp.where(a >= 0, a, NEG * a)
        a2d[r, pl.ds(c16, 16)] = a

        def mx_body(carry):
            g = plsc.load_gather(mx, [dstv])
            m = a > g
            plsc.store_scatter(mx, [dstv], a, mask=m)
            g2 = plsc.load_gather(mx, [dstv])
            return jnp.any(a > g2)
        lax.while_loop(lambda cc: cc, mx_body, True)
        return 0
    lax.fori_loop(0, N1 * 8, edge_body, 0)

    pltpu.sync_copy(mx, shared.at[s])
    plsc.subcore_barrier()
    for hh in range(2):
        pltpu.sync_copy(
            shared.at[pl.ds(0, NS), pl.ds(s * NPT + hh * (NPT // 2),
                                          NPT // 2)], buf)

        def red_body(k, _):
            c16 = pl.multiple_of(k * 16, 16)
            m = buf[0, pl.ds(c16, 16)]
            for rr in range(1, NS):
                m = jnp.maximum(m, buf[rr, pl.ds(c16, 16)])
            res[pl.ds(pl.multiple_of(hh * (NPT // 2), 16) + c16, 16)] = m
            return 0
        lax.fori_loop(0, NPT // 32, red_body, 0)
    pltpu.sync_copy(res, shared.at[NS, pl.ds(s * NPT, NPT)])
    plsc.subcore_barrier()
    pltpu.sync_copy(shared.at[NS], mx)

    lane0 = (lax.iota(jnp.int32, 16) == 0).astype(jnp.float32)

    def aexp_chunk(j, _):
        for k8 in range(8):
            c16 = pl.multiple_of(k8 * 16, 16)
            dstv = dst2d[j, pl.ds(c16, 16)]
            av = a2d[j, pl.ds(c16, 16)]
            am = plsc.load_gather(mx, [dstv])
            ae = jnp.exp(av - am)
            a2d[j, pl.ds(c16, 16)] = ae
            ax_pad[pl.ds(c16, 16)] = ae
        pltpu.sync_copy(ea_ref.at[s, j], ea_buf)

        def s2_body(i, _):
            sc = ax_pad[pl.ds(i, 16)][0]
            s2_buf[i, pl.ds(0, 16)] = ea_buf[i, pl.ds(0, 16)] * sc
            s2_buf[i, pl.ds(16, 16)] = ea_buf[i, pl.ds(16, 16)] * sc
            s2_buf[i, pl.ds(32, 16)] = lane0 * sc
            return 0
        lax.fori_loop(0, C1, s2_body, 0)
        pltpu.sync_copy(s2_buf, s2_sh.at[dst2d.at[j]], add=True)
        return 0
    lax.fori_loop(0, N1, aexp_chunk, 0)

    pltpu.sync_copy(a2d, aexp_ref.at[s])
    plsc.subcore_barrier()
    row = s * 625
    pltpu.sync_copy(s2_sh.at[pl.ds(row, 625)], s2p_ref.at[pl.ds(row, 625)])


def _k1_sc(src3, dst3, t3, s_i, s_j, ea4):
    mesh = plsc.VectorSubcoreMesh(core_axis_name="c", subcore_axis_name="s",
                                  num_cores=1)
    return pl.kernel(
        _k1_body,
        out_type=[
            jax.ShapeDtypeStruct((NS, N1, C1), jnp.float32),
            jax.ShapeDtypeStruct((N, S2W), jnp.float32),
        ],
        mesh=mesh,
        scratch_types=[
            pltpu.VMEM((NP,), jnp.float32),
            pltpu.VMEM((NP,), jnp.float32),
            pltpu.VMEM((N1, C1), jnp.int32),
            pltpu.VMEM((N1, C1), jnp.int32),
            pltpu.VMEM((N1, C1), jnp.float32),
            pltpu.VMEM((N1, C1), jnp.float32),
            pltpu.VMEM((NP,), jnp.float32),
            pltpu.VMEM((NS, NPT // 2), jnp.float32),
            pltpu.VMEM((NPT,), jnp.float32),
            pltpu.VMEM((C1 + 16,), jnp.float32),
            pltpu.VMEM((C1, EAD), jnp.float32),
            pltpu.VMEM((C1, S2W), jnp.float32),
            pltpu.VMEM_SHARED((NS + 1, NP), jnp.float32),
            pltpu.VMEM_SHARED((N, S2W), jnp.float32),
            pltpu.SemaphoreType.DMA,
        ],
        compiler_params=pltpu.CompilerParams(needs_layout_passes=False,
                                             use_tc_tiling_on_sc=False),
    )(src3, dst3, t3, s_i, s_j, ea4)


def _k2_body(src_ref, dst_ref, ax_ref, h_ref,
             aggp_ref,
             src_a, src_b, dst_c, ax1, hb_a, hb_b, agg_sh, gsa, gsb):
    wid = lax.axis_index("s")

    zv = jnp.zeros((16,), jnp.float32)

    def z_body(i, _):
        r = i // 8
        o = pl.multiple_of((i % 8) * 16, 16)
        hb_a[r, pl.ds(o, 16)] = zv
        return 0
    lax.fori_loop(0, C1 * 8, z_body, 0)
    for kk in range(5):
        row = wid * 625 + kk * 125
        pltpu.sync_copy(hb_a.at[pl.ds(0, 125)], agg_sh.at[pl.ds(row, 125)])
    plsc.subcore_barrier()

    def scale(hb, j):
        pltpu.sync_copy(ax_ref.at[wid, j], ax1.at[pl.ds(0, C1)])

        def scale_body(i, _):
            sc = ax1[pl.ds(i, 16)][0]
            for q in range(8):
                o = pl.multiple_of(q * 16, 16)
                hb[i, pl.ds(o, 16)] = hb[i, pl.ds(o, 16)] * sc
            return 0
        lax.fori_loop(0, C1, scale_body, 0)
        pltpu.sync_copy(dst_ref.at[wid, j], dst_c.at[0])
        pltpu.sync_copy(hb, agg_sh.at[dst_c.at[0]], add=True)

    pltpu.sync_copy(src_ref.at[wid, 0], src_a)
    pltpu.async_copy(h_ref.at[src_a], hb_a, gsa)

    def pair_body(j2, _):
        j = 2 * j2
        pltpu.sync_copy(src_ref.at[wid, j + 1], src_b)
        pltpu.async_copy(h_ref.at[src_b], hb_b, gsb)
        pltpu.make_async_copy(h_ref.at[src_a], hb_a, gsa).wait()
        scale(hb_a, j)
        @pl.when(j2 < N1 // 2 - 1)
        def _():
            pltpu.sync_copy(src_ref.at[wid, j + 2], src_a)
            pltpu.async_copy(h_ref.at[src_a], hb_a, gsa)
        pltpu.make_async_copy(h_ref.at[src_b], hb_b, gsb).wait()
        scale(hb_b, j + 1)
        return 0
    lax.fori_loop(0, N1 // 2, pair_body, 0)

    plsc.subcore_barrier()
    row = wid * 625
    pltpu.sync_copy(agg_sh.at[pl.ds(row, 625)],
                    aggp_ref.at[pl.ds(row, 625)])


def _k2_sc(src3, dst3, aexp3, h):
    mesh = plsc.VectorSubcoreMesh(core_axis_name="c", subcore_axis_name="s",
                                  num_cores=1)
    return pl.kernel(
        _k2_body,
        out_type=jax.ShapeDtypeStruct((N, HID), jnp.float32),
        mesh=mesh,
        scratch_types=[
            pltpu.VMEM((C1,), jnp.int32),
            pltpu.VMEM((C1,), jnp.int32),
            pltpu.VMEM((1, C1), jnp.int32),
            pltpu.VMEM((C1 + 16,), jnp.float32),
            pltpu.VMEM((C1, HID), jnp.float32),
            pltpu.VMEM((C1, HID), jnp.float32),
            pltpu.VMEM_SHARED((N, HID), jnp.float32),
            pltpu.SemaphoreType.DMA,
            pltpu.SemaphoreType.DMA,
        ],
        compiler_params=pltpu.CompilerParams(needs_layout_passes=False,
                                             use_tc_tiling_on_sc=False),
    )(src3, dst3, aexp3, h)


def _layer(x, src3, dst3, nt2, et2, edge_attr, p, relu):
    wij = p['Watt'][:2 * HID, :].reshape(2, HID, 1)[..., 0].T
    h, sij = _node_tc(x, nt2, p['Wh'], p['bh'], wij)
    ea, t = _edge_tc(edge_attr, et2, p['Wea'], p['Eemb'],
                     p['Watt'][2 * HID + ETD:, :], p['Watt'][2 * HID:2 * HID + ETD, :])
    t_p = jnp.concatenate(
        [t[:, 0], jnp.full((E_PAD - E,), -1.0e30, jnp.float32)]
    ).reshape(NS, N1, C1)
    ea4 = jnp.concatenate(
        [ea, jnp.zeros((E_PAD - E, EAD), jnp.float32)]
    ).reshape(NS, N1, C1, EAD)
    aexp3, s2p = _k1_sc(src3, dst3, t_p, sij[:, 0], sij[:, 1], ea4)
    aggp = _k2_sc(src3, dst3, aexp3, h)
    wl1 = p['Wlin'][:HID, :]
    wl2 = p['Wlin'][HID:, :]
    bl2 = p['blin'][None, :]
    return _finish_tc(h, aggp, s2p, wl1, wl2, bl2, relu)


def kernel(x, edge_index, node_type, edge_type, edge_attr, batch, params):
    src = edge_index[0].astype(jnp.int32)
    dst = edge_index[1].astype(jnp.int32)
    zpad = jnp.zeros((E_PAD - E,), jnp.int32)
    src3 = jnp.concatenate([src, zpad]).reshape(NS, N1, C1)
    dst3 = jnp.concatenate([dst, zpad]).reshape(NS, N1, C1)
    nt2 = node_type.astype(jnp.int32)[:, None]
    et2 = edge_type.astype(jnp.int32)[:, None]
    b2 = batch.astype(jnp.int32)[:, None]

    h = _layer(x, src3, dst3, nt2, et2, edge_attr, params['conv1'], True)
    h = _layer(h, src3, dst3, nt2, et2, edge_attr, params['conv2'], True)
    emb = _layer(h, src3, dst3, nt2, et2, edge_attr, params['conv3'], False)
    out = _pool_tc(emb, b2, params['W1'], params['b1'][None, :],
                   params['W2'], params['b2'][None, :])
    return emb, out
